# baseline (speedup 1.0000x reference)
import functools

import jax
import jax.numpy as jnp
from jax import lax
from jax.experimental import pallas as pl
from jax.experimental.pallas import tpu as pltpu
from jax.experimental.pallas import tpu_sc as plsc

B, S = 8, 576
N = B * S
IN_D = 768
HID = 256
OUT_D = 768

TOK_BLK = 512
N_BLOCKS = N // TOK_BLK


def _enc_argmax_table_kernel(x_ref, w_ref, b_ref, cb_ref, dw_ref, db_ref,
                             idx_ref, table_ref):
    h = lax.dot_general(x_ref[...], w_ref[...],
                        (((1,), (1,)), ((), ())),
                        preferred_element_type=jnp.float32)
    h = h + b_ref[...]
    m = jnp.max(h, axis=-1, keepdims=True)
    ii = lax.broadcasted_iota(jnp.int32, h.shape, 1)
    idx = jnp.min(jnp.where(h == m, ii, HID), axis=-1)
    idx_ref[...] = idx.astype(jnp.int32)

    @pl.when(pl.program_id(0) == 0)
    def _():
        t = lax.dot_general(cb_ref[...], dw_ref[...],
                            (((1,), (1,)), ((), ())),
                            preferred_element_type=jnp.float32)
        table_ref[...] = t + db_ref[...]


def _enc_argmax_table(xf, enc_w, enc_b2, cb, dec_w, dec_b2):
    return pl.pallas_call(
        _enc_argmax_table_kernel,
        grid=(N_BLOCKS,),
        in_specs=[
            pl.BlockSpec((TOK_BLK, IN_D), lambda i: (i, 0)),
            pl.BlockSpec((HID, IN_D), lambda i: (0, 0)),
            pl.BlockSpec((1, HID), lambda i: (0, 0)),
            pl.BlockSpec((HID, HID), lambda i: (0, 0)),
            pl.BlockSpec((OUT_D, HID), lambda i: (0, 0)),
            pl.BlockSpec((1, OUT_D), lambda i: (0, 0)),
        ],
        out_specs=[
            pl.BlockSpec((TOK_BLK,), lambda i: (i,)),
            pl.BlockSpec((HID, OUT_D), lambda i: (0, 0)),
        ],
        out_shape=[
            jax.ShapeDtypeStruct((N,), jnp.int32),
            jax.ShapeDtypeStruct((HID, OUT_D), jnp.float32),
        ],
    )(xf, enc_w, enc_b2, cb, dec_w, dec_b2)


_NC = 2
_NS = 16
_NW = _NC * _NS
_B_PER_W = N // _NW
_GCHUNK = _B_PER_W // 2


@functools.cache
def _make_sc_gather():
    @functools.partial(
        pl.kernel,
        mesh=plsc.VectorSubcoreMesh(core_axis_name="c", subcore_axis_name="s"),
        out_type=jax.ShapeDtypeStruct((N, OUT_D), jnp.float32),
        scratch_types=[
            pltpu.VMEM((_B_PER_W,), jnp.int32),
            pltpu.VMEM((_B_PER_W, OUT_D), jnp.float32),
            pltpu.SemaphoreType.DMA,
        ],
    )
    def _sc_gather(table_hbm, idx_hbm, out_hbm, idx_v, rows_v, sem):
        wid = lax.axis_index("s") * _NC + lax.axis_index("c")
        base = wid * _B_PER_W
        pltpu.sync_copy(idx_hbm.at[pl.ds(base, _B_PER_W)], idx_v)
        cps = []
        for j in range(_B_PER_W // _GCHUNK):
            cps.append(pltpu.async_copy(
                table_hbm.at[idx_v.at[pl.ds(j * _GCHUNK, _GCHUNK)]],
                rows_v.at[pl.ds(j * _GCHUNK, _GCHUNK)],
                sem))
        for cp in cps:
            cp.wait()
        pltpu.sync_copy(rows_v, out_hbm.at[pl.ds(base, _B_PER_W)])

    return _sc_gather


def kernel(x, enc_w, enc_b, dec_w, dec_b, codebook):
    xf = x.reshape(N, IN_D)
    cb = codebook[:HID]
    idx, table = _enc_argmax_table(
        xf, enc_w, enc_b.reshape(1, HID), cb, dec_w, dec_b.reshape(1, OUT_D))
    out = _make_sc_gather()(table, idx)
    return out.reshape(B, S, 1, OUT_D)


# speedup vs baseline: 1.3379x; 1.3379x over previous
import functools

import jax
import jax.numpy as jnp
from jax import lax
from jax.experimental import pallas as pl
from jax.experimental.pallas import tpu as pltpu
from jax.experimental.pallas import tpu_sc as plsc

B, S = 8, 576
N = B * S
IN_D = 768
HID = 256
OUT_D = 768

TOK_BLK = 512
N_BLOCKS = N // TOK_BLK


def _enc_argmax_table_kernel(x_ref, w_ref, b_ref, cb_ref, dw_ref, db_ref,
                             idx_ref, table_ref):
    h = lax.dot_general(x_ref[...], w_ref[...],
                        (((1,), (1,)), ((), ())),
                        preferred_element_type=jnp.float32)
    h = h + b_ref[...]
    m = jnp.max(h, axis=-1, keepdims=True)
    ii = lax.broadcasted_iota(jnp.int32, h.shape, 1)
    idx = jnp.min(jnp.where(h == m, ii, HID), axis=-1)
    idx_ref[...] = idx.astype(jnp.int32)

    @pl.when(pl.program_id(0) == 0)
    def _():
        t = lax.dot_general(cb_ref[...], dw_ref[...],
                            (((1,), (1,)), ((), ())),
                            preferred_element_type=jnp.float32)
        table_ref[...] = t + db_ref[...]


def _enc_argmax_table(xf, enc_w, enc_b2, codebook, dec_w, dec_b2):
    return pl.pallas_call(
        _enc_argmax_table_kernel,
        grid=(N_BLOCKS,),
        in_specs=[
            pl.BlockSpec((TOK_BLK, IN_D), lambda i: (i, 0)),
            pl.BlockSpec((HID, IN_D), lambda i: (0, 0)),
            pl.BlockSpec((1, HID), lambda i: (0, 0)),
            pl.BlockSpec((HID, HID), lambda i: (0, 0)),
            pl.BlockSpec((OUT_D, HID), lambda i: (0, 0)),
            pl.BlockSpec((1, OUT_D), lambda i: (0, 0)),
        ],
        out_specs=[
            pl.BlockSpec((TOK_BLK,), lambda i: (i,)),
            pl.BlockSpec((HID, OUT_D), lambda i: (0, 0)),
        ],
        out_shape=[
            jax.ShapeDtypeStruct((N,), jnp.int32),
            jax.ShapeDtypeStruct((HID, OUT_D), jnp.float32),
        ],
    )(xf, enc_w, enc_b2, codebook, dec_w, dec_b2)


_NC = 2
_NS = 16
_NW = _NC * _NS
_B_PER_W = N // _NW
_GCHUNK = _B_PER_W // 2


@functools.cache
def _make_sc_gather():
    @functools.partial(
        pl.kernel,
        mesh=plsc.VectorSubcoreMesh(core_axis_name="c", subcore_axis_name="s"),
        out_type=jax.ShapeDtypeStruct((N, 1, OUT_D), jnp.float32),
        scratch_types=[
            pltpu.VMEM((_B_PER_W,), jnp.int32),
            pltpu.VMEM((_B_PER_W, 1, OUT_D), jnp.float32),
            pltpu.SemaphoreType.DMA,
        ],
    )
    def _sc_gather(table_hbm, idx_hbm, out_hbm, idx_v, rows_v, sem):
        wid = lax.axis_index("s") * _NC + lax.axis_index("c")
        base = wid * _B_PER_W
        pltpu.sync_copy(idx_hbm.at[pl.ds(base, _B_PER_W)], idx_v)
        cps = []
        for j in range(_B_PER_W // _GCHUNK):
            cps.append(pltpu.async_copy(
                table_hbm.at[idx_v.at[pl.ds(j * _GCHUNK, _GCHUNK)]],
                rows_v.at[pl.ds(j * _GCHUNK, _GCHUNK)],
                sem))
        for cp in cps:
            cp.wait()
        pltpu.sync_copy(rows_v, out_hbm.at[pl.ds(base, _B_PER_W)])

    return _sc_gather


def kernel(x, enc_w, enc_b, dec_w, dec_b, codebook):
    xf = x.reshape(N, IN_D)
    idx, table = _enc_argmax_table(
        xf, enc_w, enc_b.reshape(1, HID), codebook, dec_w,
        dec_b.reshape(1, OUT_D))
    out = _make_sc_gather()(table.reshape(HID, 1, OUT_D), idx)
    return out.reshape(B, S, 1, OUT_D)


# speedup vs baseline: 1.3978x; 1.0448x over previous
import functools

import jax
import jax.numpy as jnp
from jax import lax
from jax.experimental import pallas as pl
from jax.experimental.pallas import tpu as pltpu
from jax.experimental.pallas import tpu_sc as plsc

B, S = 8, 576
N = B * S
IN_D = 768
HID = 256
OUT_D = 768

TOK_BLK = 512
N_BLOCKS = N // TOK_BLK


def _enc_argmax_kernel(x_ref, w_ref, b_ref, idx_ref):
    ht = lax.dot_general(w_ref[...], x_ref[...],
                         (((1,), (1,)), ((), ())),
                         preferred_element_type=jnp.float32)
    ht = ht + b_ref[...]
    m = jnp.max(ht, axis=0, keepdims=True)
    ii = lax.broadcasted_iota(jnp.int32, ht.shape, 0)
    idx = jnp.min(jnp.where(ht == m, ii, HID), axis=0)
    idx_ref[...] = idx.astype(jnp.int32)


def _enc_argmax(xf, enc_w, enc_bc):
    return pl.pallas_call(
        _enc_argmax_kernel,
        grid=(N_BLOCKS,),
        in_specs=[
            pl.BlockSpec((TOK_BLK, IN_D), lambda i: (i, 0)),
            pl.BlockSpec((HID, IN_D), lambda i: (0, 0)),
            pl.BlockSpec((HID, 1), lambda i: (0, 0)),
        ],
        out_specs=pl.BlockSpec((TOK_BLK,), lambda i: (i,)),
        out_shape=jax.ShapeDtypeStruct((N,), jnp.int32),
    )(xf, enc_w, enc_bc)


def _table_kernel(cb_ref, dw_ref, db_ref, table_ref):
    t = lax.dot_general(cb_ref[...], dw_ref[...],
                        (((1,), (1,)), ((), ())),
                        preferred_element_type=jnp.float32)
    table_ref[...] = (t + db_ref[...])[:, None, :]


def _make_table(codebook, dec_w, dec_b2):
    return pl.pallas_call(
        _table_kernel,
        grid=(1,),
        in_specs=[
            pl.BlockSpec((HID, HID), lambda i: (0, 0)),
            pl.BlockSpec((OUT_D, HID), lambda i: (0, 0)),
            pl.BlockSpec((1, OUT_D), lambda i: (0, 0)),
        ],
        out_specs=pl.BlockSpec((HID, 1, OUT_D), lambda i: (0, 0, 0)),
        out_shape=jax.ShapeDtypeStruct((HID, 1, OUT_D), jnp.float32),
    )(codebook, dec_w, dec_b2)


_NC = 2
_NS = 16
_NW = _NC * _NS
_B_PER_W = N // _NW
_GCHUNK = _B_PER_W // 2


@functools.cache
def _make_sc_gather():
    @functools.partial(
        pl.kernel,
        mesh=plsc.VectorSubcoreMesh(core_axis_name="c", subcore_axis_name="s"),
        out_type=jax.ShapeDtypeStruct((N, 1, OUT_D), jnp.float32),
        scratch_types=[
            pltpu.VMEM((_B_PER_W,), jnp.int32),
            pltpu.VMEM((_B_PER_W, 1, OUT_D), jnp.float32),
            pltpu.SemaphoreType.DMA,
        ],
    )
    def _sc_gather(table_hbm, idx_hbm, out_hbm, idx_v, rows_v, sem):
        wid = lax.axis_index("s") * _NC + lax.axis_index("c")
        base = wid * _B_PER_W
        pltpu.sync_copy(idx_hbm.at[pl.ds(base, _B_PER_W)], idx_v)
        cps = []
        for j in range(_B_PER_W // _GCHUNK):
            cps.append(pltpu.async_copy(
                table_hbm.at[idx_v.at[pl.ds(j * _GCHUNK, _GCHUNK)]],
                rows_v.at[pl.ds(j * _GCHUNK, _GCHUNK)],
                sem))
        for cp in cps:
            cp.wait()
        pltpu.sync_copy(rows_v, out_hbm.at[pl.ds(base, _B_PER_W)])

    return _sc_gather


def kernel(x, enc_w, enc_b, dec_w, dec_b, codebook):
    xf = x.reshape(N, IN_D)
    table = _make_table(codebook, dec_w, dec_b.reshape(1, OUT_D))
    idx = _enc_argmax(xf, enc_w, enc_b.reshape(HID, 1))
    out = _make_sc_gather()(table, idx)
    return out.reshape(B, S, 1, OUT_D)


# speedup vs baseline: 1.5414x; 1.1027x over previous
import functools

import jax
import jax.numpy as jnp
from jax import lax
from jax.experimental import pallas as pl
from jax.experimental.pallas import tpu as pltpu
from jax.experimental.pallas import tpu_sc as plsc

B, S = 8, 576
N = B * S
IN_D = 768
HID = 256
OUT_D = 768

TOK_BLK = 512
N_BLOCKS = N // TOK_BLK


def _enc_argmax_table_kernel(x_ref, w_ref, b_ref, cb_ref, dw_ref, db_ref,
                             idx_ref, table_ref):
    ht = lax.dot_general(w_ref[...], x_ref[...],
                         (((1,), (1,)), ((), ())),
                         preferred_element_type=jnp.float32)
    ht = ht + jnp.transpose(jnp.reshape(b_ref[...], (1, HID)))
    m = jnp.max(ht, axis=0, keepdims=True)
    ii = lax.broadcasted_iota(jnp.int32, ht.shape, 0)
    idx = jnp.min(jnp.where(ht == m, ii, HID), axis=0)
    idx_ref[...] = idx.astype(jnp.int32)

    @pl.when(pl.program_id(0) == 0)
    def _():
        t = lax.dot_general(cb_ref[...], dw_ref[...],
                            (((1,), (1,)), ((), ())),
                            preferred_element_type=jnp.float32)
        table_ref[...] = (t + db_ref[...][None, :])[:, None, :]


def _enc_argmax_table(xf, enc_w, enc_b, codebook, dec_w, dec_b):
    return pl.pallas_call(
        _enc_argmax_table_kernel,
        grid=(N_BLOCKS,),
        in_specs=[
            pl.BlockSpec((TOK_BLK, IN_D), lambda i: (i, 0)),
            pl.BlockSpec((HID, IN_D), lambda i: (0, 0)),
            pl.BlockSpec((HID,), lambda i: (0,)),
            pl.BlockSpec((HID, HID), lambda i: (0, 0)),
            pl.BlockSpec((OUT_D, HID), lambda i: (0, 0)),
            pl.BlockSpec((OUT_D,), lambda i: (0,)),
        ],
        out_specs=[
            pl.BlockSpec((TOK_BLK,), lambda i: (i,)),
            pl.BlockSpec((HID, 1, OUT_D), lambda i: (0, 0, 0)),
        ],
        out_shape=[
            jax.ShapeDtypeStruct((N,), jnp.int32),
            jax.ShapeDtypeStruct((HID, 1, OUT_D), jnp.float32),
        ],
    )(xf, enc_w, enc_b, codebook, dec_w, dec_b)


_NC = 2
_NS = 16
_NW = _NC * _NS
_B_PER_W = N // _NW
_GCHUNK = 48
_NCHUNK = _B_PER_W // _GCHUNK


@functools.cache
def _make_sc_gather():
    @functools.partial(
        pl.kernel,
        mesh=plsc.VectorSubcoreMesh(core_axis_name="c", subcore_axis_name="s"),
        out_type=jax.ShapeDtypeStruct((N, 1, OUT_D), jnp.float32),
        scratch_types=(
            [pltpu.VMEM((_B_PER_W,), jnp.int32),
             pltpu.VMEM((_B_PER_W, 1, OUT_D), jnp.float32)]
            + [pltpu.SemaphoreType.DMA] * (2 * _NCHUNK)
        ),
    )
    def _sc_gather(table_hbm, idx_hbm, out_hbm, idx_v, rows_v, *sems):
        wid = lax.axis_index("s") * _NC + lax.axis_index("c")
        base = wid * _B_PER_W
        pltpu.sync_copy(idx_hbm.at[pl.ds(base, _B_PER_W)], idx_v)
        gathers = []
        for j in range(_NCHUNK):
            gathers.append(pltpu.async_copy(
                table_hbm.at[idx_v.at[pl.ds(j * _GCHUNK, _GCHUNK)]],
                rows_v.at[pl.ds(j * _GCHUNK, _GCHUNK)],
                sems[j]))
        scatters = []
        for j in range(_NCHUNK):
            gathers[j].wait()
            scatters.append(pltpu.async_copy(
                rows_v.at[pl.ds(j * _GCHUNK, _GCHUNK)],
                out_hbm.at[pl.ds(base + j * _GCHUNK, _GCHUNK)],
                sems[_NCHUNK + j]))
        for cp in scatters:
            cp.wait()

    return _sc_gather


def kernel(x, enc_w, enc_b, dec_w, dec_b, codebook):
    xf = x.reshape(N, IN_D)
    idx, table = _enc_argmax_table(xf, enc_w, enc_b, codebook, dec_w, dec_b)
    out = _make_sc_gather()(table, idx)
    return out.reshape(B, S, 1, OUT_D)
